# baseline (speedup 1.0000x reference)
import functools

import jax
import jax.numpy as jnp
from jax import lax
from jax.experimental import pallas as pl
from jax.experimental.pallas import tpu as pltpu
from jax.experimental.pallas import tpu_sc as plsc

EMB = 64
SEQ = 200
NB = 4096
VOCAB = 1000000
G = 128
L = 16
NC = 2
NS = 16
NW = NC * NS

NBLK = VOCAB // G
TAIL = VOCAB - NBLK * G

_PARAMS = pltpu.CompilerParams(
    use_tc_tiling_on_sc=True, needs_layout_passes=False
)


def _pack(tab_t, tail):
    mesh = plsc.VectorSubcoreMesh(core_axis_name="c", subcore_axis_name="s")

    @functools.partial(
        pl.kernel,
        out_type=jax.ShapeDtypeStruct((VOCAB // 2, 2 * EMB), jnp.float32),
        mesh=mesh,
        compiler_params=_PARAMS,
        scratch_types=[
            pltpu.VMEM((EMB, G), jnp.float32),
            pltpu.VMEM((G // 2, 2 * EMB), jnp.float32),
            pltpu.VMEM((TAIL, EMB), jnp.float32),
        ],
    )
    def body(t_hbm, tail_hbm, out_hbm, blk_v, pck_v, tail_v):
        wid = lax.axis_index("s") * NC + lax.axis_index("c")
        iota = lax.iota(jnp.int32, L)
        nblk_w = (NBLK - wid + NW - 1) // NW

        def blk_body(i, carry):
            j = wid + NW * i
            pltpu.sync_copy(
                t_hbm.at[:, pl.ds(pl.multiple_of(j * G, G), G)], blk_v
            )

            def q_body(q, carry2):
                col0 = jnp.full((L,), 2 * q, jnp.int32)
                for d in range(2 * EMB // L):
                    rows = (d % (EMB // L)) * L + iota
                    val = plsc.load_gather(blk_v, [rows, col0 + d // (EMB // L)])
                    pck_v[q, pl.ds(d * L, L)] = val
                return carry2

            lax.fori_loop(0, G // 2, q_body, 0)
            pltpu.sync_copy(
                pck_v, out_hbm.at[pl.ds(pl.multiple_of(j * (G // 2), 8), G // 2)]
            )
            return carry

        lax.fori_loop(0, nblk_w, blk_body, 0)

        @pl.when(wid == 0)
        def _():
            pltpu.sync_copy(tail_hbm, tail_v)

            def t_body(q, carry2):
                for d in range(2 * EMB // L):
                    r = 2 * q + d // (EMB // L)
                    pck_v[q, pl.ds(d * L, L)] = tail_v[
                        r, pl.ds((d % (EMB // L)) * L, L)
                    ]
                return carry2

            lax.fori_loop(0, TAIL // 2, t_body, 0)
            pltpu.sync_copy(
                pck_v.at[pl.ds(0, TAIL // 2)],
                out_hbm.at[pl.ds(NBLK * G // 2, TAIL // 2)],
            )

    return body(tab_t, tail)


def _gather(pck, x2, pe):
    mesh = plsc.VectorSubcoreMesh(core_axis_name="c", subcore_axis_name="s")

    @functools.partial(
        pl.kernel,
        out_type=jax.ShapeDtypeStruct((SEQ, EMB, NB), jnp.float32),
        mesh=mesh,
        compiler_params=_PARAMS,
        scratch_types=[
            pltpu.VMEM((SEQ, G), jnp.int32),
            pltpu.VMEM((SEQ, EMB), jnp.float32),
            pltpu.VMEM((G,), jnp.int32),
            pltpu.VMEM((G,), jnp.int32),
            pltpu.VMEM((G, G), jnp.float32),
            pltpu.VMEM((EMB, G), jnp.float32),
            pltpu.SemaphoreType.DMA,
        ],
    )
    def body(tab_hbm, x_hbm, pe_hbm, out_hbm, xb_v, pe_v, idx_v, par_v, g_v, cell_v, sem):
        wid = lax.axis_index("s") * NC + lax.axis_index("c")
        pltpu.sync_copy(pe_hbm, pe_v)
        pltpu.sync_copy(x_hbm.at[pl.ds(pl.multiple_of(wid * SEQ, 8), SEQ)], xb_v)
        iota = lax.iota(jnp.int32, L)

        def cell_body(s, carry):
            for k in range(G // L):
                u = (s + SEQ * L * k) + SEQ * iota
                v = plsc.load_gather(xb_v, [u >> 7, u & 127])
                idx_v[pl.ds(k * L, L)] = v >> 1
                par_v[pl.ds(k * L, L)] = (v & 1) << 6
            pltpu.async_copy(tab_hbm.at[idx_v], g_v, sem).wait()

            def k_body(k, carry2):
                rows = k * L + iota
                par64 = par_v[pl.ds(k * L, L)]
                for e in range(EMB):
                    sp = plsc.load_gather(
                        pe_v, [jnp.full((L,), s, jnp.int32),
                               jnp.full((L,), e, jnp.int32)]
                    )
                    val = plsc.load_gather(g_v, [rows, par64 + e])
                    cell_v[e, pl.ds(k * L, L)] = val + sp
                return carry2

            lax.fori_loop(0, G // L, k_body, 0)
            pltpu.sync_copy(
                cell_v, out_hbm.at[s, :, pl.ds(pl.multiple_of(wid * G, G), G)]
            )
            return carry

        lax.fori_loop(0, SEQ, cell_body, 0)

    return body(pck, x2, pe)


def kernel(x, table, pos_enc):
    batch, seq = x.shape
    x2 = x.reshape(batch * seq // G, G)
    pck = _pack(table.T, table[NBLK * G :, :])
    out3 = _gather(pck, x2, pos_enc[:seq])
    return out3.transpose(2, 0, 1)


# speedup vs baseline: 1.0082x; 1.0082x over previous
import functools

import jax
import jax.numpy as jnp
from jax import lax
from jax.experimental import pallas as pl
from jax.experimental.pallas import tpu as pltpu
from jax.experimental.pallas import tpu_sc as plsc

EMB = 64
SEQ = 200
NB = 4096
VOCAB = 1000000
G = 128
L = 16
NC = 2
NS = 16
NW = NC * NS

NBLK = VOCAB // G
TAIL = VOCAB - NBLK * G

_PARAMS = pltpu.CompilerParams(
    use_tc_tiling_on_sc=True, needs_layout_passes=False
)


def _pack(tab_t, tail):
    mesh = plsc.VectorSubcoreMesh(core_axis_name="c", subcore_axis_name="s")

    @functools.partial(
        pl.kernel,
        out_type=jax.ShapeDtypeStruct((VOCAB // 2, 2 * EMB), jnp.float32),
        mesh=mesh,
        compiler_params=_PARAMS,
        scratch_types=[
            pltpu.VMEM((EMB, G), jnp.float32),
            pltpu.VMEM((G // 2, 2 * EMB), jnp.float32),
            pltpu.VMEM((TAIL, EMB), jnp.float32),
        ],
    )
    def body(t_hbm, tail_hbm, out_hbm, blk_v, pck_v, tail_v):
        wid = lax.axis_index("s") * NC + lax.axis_index("c")
        iota = lax.iota(jnp.int32, L)
        nblk_w = (NBLK - wid + NW - 1) // NW

        def blk_body(i, carry):
            j = wid + NW * i
            pltpu.sync_copy(
                t_hbm.at[:, pl.ds(pl.multiple_of(j * G, G), G)], blk_v
            )

            rows_l = [d * L + iota for d in range(EMB // L)]
            for q in range(G // 2):
                for d in range(2 * EMB // L):
                    col = jnp.full((L,), 2 * q + d // (EMB // L), jnp.int32)
                    val = plsc.load_gather(blk_v, [rows_l[d % (EMB // L)], col])
                    pck_v[q, pl.ds(d * L, L)] = val
            pltpu.sync_copy(
                pck_v, out_hbm.at[pl.ds(pl.multiple_of(j * (G // 2), 8), G // 2)]
            )
            return carry

        lax.fori_loop(0, nblk_w, blk_body, 0)

        @pl.when(wid == 0)
        def _():
            pltpu.sync_copy(tail_hbm, tail_v)

            def t_body(q, carry2):
                for d in range(2 * EMB // L):
                    r = 2 * q + d // (EMB // L)
                    pck_v[q, pl.ds(d * L, L)] = tail_v[
                        r, pl.ds((d % (EMB // L)) * L, L)
                    ]
                return carry2

            lax.fori_loop(0, TAIL // 2, t_body, 0)
            pltpu.sync_copy(
                pck_v.at[pl.ds(0, TAIL // 2)],
                out_hbm.at[pl.ds(NBLK * G // 2, TAIL // 2)],
            )

    return body(tab_t, tail)


def _gather(pck, x2, pe):
    mesh = plsc.VectorSubcoreMesh(core_axis_name="c", subcore_axis_name="s")

    @functools.partial(
        pl.kernel,
        out_type=jax.ShapeDtypeStruct((SEQ, EMB, NB), jnp.float32),
        mesh=mesh,
        compiler_params=_PARAMS,
        scratch_types=[
            pltpu.VMEM((SEQ, G), jnp.int32),
            pltpu.VMEM((SEQ, EMB), jnp.float32),
            pltpu.VMEM((G,), jnp.int32),
            pltpu.VMEM((G,), jnp.int32),
            pltpu.VMEM((G, G), jnp.float32),
            pltpu.VMEM((EMB, G), jnp.float32),
            pltpu.SemaphoreType.DMA,
        ],
    )
    def body(tab_hbm, x_hbm, pe_hbm, out_hbm, xb_v, pe_v, idx_v, par_v, g_v, cell_v, sem):
        wid = lax.axis_index("s") * NC + lax.axis_index("c")
        pltpu.sync_copy(pe_hbm, pe_v)
        pltpu.sync_copy(x_hbm.at[pl.ds(pl.multiple_of(wid * SEQ, 8), SEQ)], xb_v)
        iota = lax.iota(jnp.int32, L)

        def cell_body(s, carry):
            for k in range(G // L):
                u = (s + SEQ * L * k) + SEQ * iota
                v = plsc.load_gather(xb_v, [u >> 7, u & 127])
                idx_v[pl.ds(k * L, L)] = v >> 1
                par_v[pl.ds(k * L, L)] = (v & 1) << 6
            pltpu.async_copy(tab_hbm.at[idx_v], g_v, sem).wait()

            s_vec = jnp.full((L,), s, jnp.int32)
            rows_l = [k * L + iota for k in range(G // L)]
            par_l = [par_v[pl.ds(k * L, L)] for k in range(G // L)]
            for e in range(EMB):
                sp = plsc.load_gather(pe_v, [s_vec, jnp.full((L,), e, jnp.int32)])
                for k in range(G // L):
                    val = plsc.load_gather(g_v, [rows_l[k], par_l[k] + e])
                    cell_v[e, pl.ds(k * L, L)] = val + sp
            pltpu.sync_copy(
                cell_v, out_hbm.at[s, :, pl.ds(pl.multiple_of(wid * G, G), G)]
            )
            return carry

        lax.fori_loop(0, SEQ, cell_body, 0)

    return body(pck, x2, pe)


def kernel(x, table, pos_enc):
    batch, seq = x.shape
    x2 = x.reshape(batch * seq // G, G)
    pck = _pack(table.T, table[NBLK * G :, :])
    out3 = _gather(pck, x2, pos_enc[:seq])
    return out3.transpose(2, 0, 1)


# speedup vs baseline: 1.6566x; 1.6432x over previous
import functools

import jax
import jax.numpy as jnp
from jax import lax
from jax.experimental import pallas as pl
from jax.experimental.pallas import tpu as pltpu
from jax.experimental.pallas import tpu_sc as plsc

EMB = 64
SEQ = 200
NB = 4096
VOCAB = 1000000
G = 128
L = 16
NC = 2
NS = 16
NW = NC * NS

NBLK = VOCAB // G
TAIL = VOCAB - NBLK * G

_PARAMS = pltpu.CompilerParams(
    use_tc_tiling_on_sc=True, needs_layout_passes=False
)


def _pack(tab_t, tail):
    mesh = plsc.VectorSubcoreMesh(core_axis_name="c", subcore_axis_name="s")

    @functools.partial(
        pl.kernel,
        out_type=jax.ShapeDtypeStruct((VOCAB // 2, 2 * EMB), jnp.float32),
        mesh=mesh,
        compiler_params=_PARAMS,
        scratch_types=[
            pltpu.VMEM((EMB, G), jnp.float32),
            pltpu.VMEM((G // 2, 2 * EMB), jnp.float32),
            pltpu.VMEM((TAIL, EMB), jnp.float32),
        ],
    )
    def body(t_hbm, tail_hbm, out_hbm, blk_v, pck_v, tail_v):
        wid = lax.axis_index("s") * NC + lax.axis_index("c")
        iota = lax.iota(jnp.int32, L)
        nblk_w = (NBLK - wid + NW - 1) // NW

        def blk_body(i, carry):
            j = wid + NW * i
            pltpu.sync_copy(
                t_hbm.at[:, pl.ds(pl.multiple_of(j * G, G), G)], blk_v
            )

            rows_l = [d * L + iota for d in range(EMB // L)]

            @plsc.parallel_loop(0, G // 2, 1, unroll=4)
            def _(q):
                for d in range(2 * EMB // L):
                    col = jnp.full((L,), 2 * q + d // (EMB // L), jnp.int32)
                    val = plsc.load_gather(blk_v, [rows_l[d % (EMB // L)], col])
                    pck_v[q, pl.ds(d * L, L)] = val
            pltpu.sync_copy(
                pck_v, out_hbm.at[pl.ds(pl.multiple_of(j * (G // 2), 8), G // 2)]
            )
            return carry

        lax.fori_loop(0, nblk_w, blk_body, 0)

        @pl.when(wid == 0)
        def _():
            pltpu.sync_copy(tail_hbm, tail_v)

            def t_body(q, carry2):
                for d in range(2 * EMB // L):
                    r = 2 * q + d // (EMB // L)
                    pck_v[q, pl.ds(d * L, L)] = tail_v[
                        r, pl.ds((d % (EMB // L)) * L, L)
                    ]
                return carry2

            lax.fori_loop(0, TAIL // 2, t_body, 0)
            pltpu.sync_copy(
                pck_v.at[pl.ds(0, TAIL // 2)],
                out_hbm.at[pl.ds(NBLK * G // 2, TAIL // 2)],
            )

    return body(tab_t, tail)


def _gather(pck, x2, pe):
    mesh = plsc.VectorSubcoreMesh(core_axis_name="c", subcore_axis_name="s")

    @functools.partial(
        pl.kernel,
        out_type=jax.ShapeDtypeStruct((SEQ, EMB, NB), jnp.float32),
        mesh=mesh,
        compiler_params=_PARAMS,
        scratch_types=[
            pltpu.VMEM((SEQ, G), jnp.int32),
            pltpu.VMEM((SEQ, EMB), jnp.float32),
            pltpu.VMEM((G,), jnp.int32),
            pltpu.VMEM((G,), jnp.int32),
            pltpu.VMEM((G, G), jnp.float32),
            pltpu.VMEM((EMB, G), jnp.float32),
            pltpu.SemaphoreType.DMA,
        ],
    )
    def body(tab_hbm, x_hbm, pe_hbm, out_hbm, xb_v, pe_v, idx_v, par_v, g_v, cell_v, sem):
        wid = lax.axis_index("s") * NC + lax.axis_index("c")
        pltpu.sync_copy(pe_hbm, pe_v)
        pltpu.sync_copy(x_hbm.at[pl.ds(pl.multiple_of(wid * SEQ, 8), SEQ)], xb_v)
        iota = lax.iota(jnp.int32, L)

        def cell_body(s, carry):
            for k in range(G // L):
                u = (s + SEQ * L * k) + SEQ * iota
                v = plsc.load_gather(xb_v, [u >> 7, u & 127])
                idx_v[pl.ds(k * L, L)] = v >> 1
                par_v[pl.ds(k * L, L)] = (v & 1) << 6
            pltpu.async_copy(tab_hbm.at[idx_v], g_v, sem).wait()

            s_vec = jnp.full((L,), s, jnp.int32)
            rows_l = [k * L + iota for k in range(G // L)]
            par_l = [par_v[pl.ds(k * L, L)] for k in range(G // L)]

            @plsc.parallel_loop(0, EMB, 1, unroll=8)
            def _(e):
                sp = plsc.load_gather(pe_v, [s_vec, jnp.full((L,), e, jnp.int32)])
                for k in range(G // L):
                    val = plsc.load_gather(g_v, [rows_l[k], par_l[k] + e])
                    cell_v[e, pl.ds(k * L, L)] = val + sp
            pltpu.sync_copy(
                cell_v, out_hbm.at[s, :, pl.ds(pl.multiple_of(wid * G, G), G)]
            )
            return carry

        lax.fori_loop(0, SEQ, cell_body, 0)

    return body(pck, x2, pe)


def kernel(x, table, pos_enc):
    batch, seq = x.shape
    x2 = x.reshape(batch * seq // G, G)
    pck = _pack(table.T, table[NBLK * G :, :])
    out3 = _gather(pck, x2, pos_enc[:seq])
    return out3.transpose(2, 0, 1)


# speedup vs baseline: 3.4582x; 2.0875x over previous
import functools

import jax
import jax.numpy as jnp
from jax import lax
from jax.experimental import pallas as pl
from jax.experimental.pallas import tpu as pltpu
from jax.experimental.pallas import tpu_sc as plsc

EMB = 64
SEQ = 200
NB = 4096
VOCAB = 1000000
G = 128
L = 16
NC = 2
NS = 16
NW = NC * NS

NBLK = VOCAB // G
TAIL = VOCAB - NBLK * G

_PARAMS = pltpu.CompilerParams(
    use_tc_tiling_on_sc=True, needs_layout_passes=False
)


def _pack(tab_t, tail):
    mesh = plsc.VectorSubcoreMesh(core_axis_name="c", subcore_axis_name="s")

    @functools.partial(
        pl.kernel,
        out_type=jax.ShapeDtypeStruct((VOCAB // 2, 2 * EMB), jnp.float32),
        mesh=mesh,
        compiler_params=_PARAMS,
        scratch_types=[
            pltpu.VMEM((EMB, G), jnp.float32),
            pltpu.VMEM((G // 2, 2 * EMB), jnp.float32),
            pltpu.VMEM((TAIL, EMB), jnp.float32),
        ],
    )
    def body(t_hbm, tail_hbm, out_hbm, blk_v, pck_v, tail_v):
        wid = lax.axis_index("s") * NC + lax.axis_index("c")
        iota = lax.iota(jnp.int32, L)
        nblk_w = (NBLK - wid + NW - 1) // NW

        def blk_body(i, carry):
            j = wid + NW * i
            pltpu.sync_copy(
                t_hbm.at[:, pl.ds(pl.multiple_of(j * G, G), G)], blk_v
            )

            cols2 = 2 * iota

            @plsc.parallel_loop(0, L, 1, unroll=2)
            def _(t):
                perm = (iota + t) & (L - 1)
                for ei in range(EMB // L):
                    evec = perm + ei * L
                    for qi in range(G // 2 // L):
                        rows_q = qi * L + iota
                        for h in range(2):
                            val = plsc.load_gather(
                                blk_v, [evec, cols2 + (2 * qi * L + h)]
                            )
                            plsc.store_scatter(
                                pck_v, [rows_q, evec + h * EMB], val
                            )
            pltpu.sync_copy(
                pck_v, out_hbm.at[pl.ds(pl.multiple_of(j * (G // 2), 8), G // 2)]
            )
            return carry

        lax.fori_loop(0, nblk_w, blk_body, 0)

        @pl.when(wid == 0)
        def _():
            pltpu.sync_copy(tail_hbm, tail_v)

            def t_body(q, carry2):
                for d in range(2 * EMB // L):
                    r = 2 * q + d // (EMB // L)
                    pck_v[q, pl.ds(d * L, L)] = tail_v[
                        r, pl.ds((d % (EMB // L)) * L, L)
                    ]
                return carry2

            lax.fori_loop(0, TAIL // 2, t_body, 0)
            pltpu.sync_copy(
                pck_v.at[pl.ds(0, TAIL // 2)],
                out_hbm.at[pl.ds(NBLK * G // 2, TAIL // 2)],
            )

    return body(tab_t, tail)


def _gather(pck, x2, pe):
    mesh = plsc.VectorSubcoreMesh(core_axis_name="c", subcore_axis_name="s")

    @functools.partial(
        pl.kernel,
        out_type=jax.ShapeDtypeStruct((SEQ, EMB, NB), jnp.float32),
        mesh=mesh,
        compiler_params=_PARAMS,
        scratch_types=[
            pltpu.VMEM((SEQ, G), jnp.int32),
            pltpu.VMEM((SEQ, EMB), jnp.float32),
            pltpu.VMEM((G,), jnp.int32),
            pltpu.VMEM((G,), jnp.int32),
            pltpu.VMEM((G, G), jnp.float32),
            pltpu.VMEM((EMB, G), jnp.float32),
            pltpu.SemaphoreType.DMA,
        ],
    )
    def body(tab_hbm, x_hbm, pe_hbm, out_hbm, xb_v, pe_v, idx_v, par_v, g_v, cell_v, sem):
        wid = lax.axis_index("s") * NC + lax.axis_index("c")
        pltpu.sync_copy(pe_hbm, pe_v)
        pltpu.sync_copy(x_hbm.at[pl.ds(pl.multiple_of(wid * SEQ, 8), SEQ)], xb_v)
        iota = lax.iota(jnp.int32, L)

        def cell_body(s, carry):
            for k in range(G // L):
                u = (s + SEQ * L * k) + SEQ * iota
                v = plsc.load_gather(xb_v, [u >> 7, u & 127])
                idx_v[pl.ds(k * L, L)] = v >> 1
                par_v[pl.ds(k * L, L)] = (v & 1) << 6
            pltpu.async_copy(tab_hbm.at[idx_v], g_v, sem).wait()

            cols_l = [k * L + iota for k in range(G // L)]
            par_l = [par_v[pl.ds(k * L, L)] for k in range(G // L)]
            pe_row = [pe_v[s, pl.ds(ei * L, L)] for ei in range(EMB // L)]

            @plsc.parallel_loop(0, L, 1, unroll=2)
            def _(t):
                perm = (iota + t) & (L - 1)
                for ei in range(EMB // L):
                    evec = perm + ei * L
                    sp = jnp.take_along_axis(pe_row[ei], perm, axis=0)
                    for ci in range(G // L):
                        val = plsc.load_gather(g_v, [cols_l[ci], par_l[ci] + evec])
                        plsc.store_scatter(cell_v, [evec, cols_l[ci]], val + sp)
            pltpu.sync_copy(
                cell_v, out_hbm.at[s, :, pl.ds(pl.multiple_of(wid * G, G), G)]
            )
            return carry

        lax.fori_loop(0, SEQ, cell_body, 0)

    return body(pck, x2, pe)


def kernel(x, table, pos_enc):
    batch, seq = x.shape
    x2 = x.reshape(batch * seq // G, G)
    pck = _pack(table.T, table[NBLK * G :, :])
    out3 = _gather(pck, x2, pos_enc[:seq])
    return out3.transpose(2, 0, 1)


# speedup vs baseline: 4.2344x; 1.2245x over previous
import functools

import jax
import jax.numpy as jnp
from jax import lax
from jax.experimental import pallas as pl
from jax.experimental.pallas import tpu as pltpu
from jax.experimental.pallas import tpu_sc as plsc

EMB = 64
SEQ = 200
NB = 4096
VOCAB = 1000000
G = 128
L = 16
NC = 2
NS = 16
NW = NC * NS

NBLK = VOCAB // G
TAIL = VOCAB - NBLK * G

_PARAMS = pltpu.CompilerParams(
    use_tc_tiling_on_sc=True, needs_layout_passes=False
)


def _pack(tab_t, tail):
    mesh = plsc.VectorSubcoreMesh(core_axis_name="c", subcore_axis_name="s")

    @functools.partial(
        pl.kernel,
        out_type=jax.ShapeDtypeStruct((VOCAB // 2, 2 * EMB), jnp.float32),
        mesh=mesh,
        compiler_params=_PARAMS,
        scratch_types=[
            pltpu.VMEM((EMB, G), jnp.float32),
            pltpu.VMEM((G // 2, 2 * EMB), jnp.float32),
            pltpu.VMEM((TAIL, EMB), jnp.float32),
        ],
    )
    def body(t_hbm, tail_hbm, out_hbm, blk_v, pck_v, tail_v):
        wid = lax.axis_index("s") * NC + lax.axis_index("c")
        iota = lax.iota(jnp.int32, L)
        nblk_w = (NBLK - wid + NW - 1) // NW

        def blk_body(i, carry):
            j = wid + NW * i
            pltpu.sync_copy(
                t_hbm.at[:, pl.ds(pl.multiple_of(j * G, G), G)], blk_v
            )

            cols_l = [cg * L + iota for cg in range(G // L)]
            rows_h = [(cg * L + iota) >> 1 for cg in range(G // L)]
            hvec = (iota & 1) * EMB

            @plsc.parallel_loop(0, L, 1, unroll=4)
            def _(t):
                perm = (iota + t) & (L - 1)
                for ei in range(EMB // L):
                    evec = perm + ei * L
                    ecol = evec + hvec
                    for cg in range(G // L):
                        val = plsc.load_gather(blk_v, [evec, cols_l[cg]])
                        plsc.store_scatter(pck_v, [rows_h[cg], ecol], val)
            pltpu.sync_copy(
                pck_v, out_hbm.at[pl.ds(pl.multiple_of(j * (G // 2), 8), G // 2)]
            )
            return carry

        lax.fori_loop(0, nblk_w, blk_body, 0)

        @pl.when(wid == 0)
        def _():
            pltpu.sync_copy(tail_hbm, tail_v)

            def t_body(q, carry2):
                for d in range(2 * EMB // L):
                    r = 2 * q + d // (EMB // L)
                    pck_v[q, pl.ds(d * L, L)] = tail_v[
                        r, pl.ds((d % (EMB // L)) * L, L)
                    ]
                return carry2

            lax.fori_loop(0, TAIL // 2, t_body, 0)
            pltpu.sync_copy(
                pck_v.at[pl.ds(0, TAIL // 2)],
                out_hbm.at[pl.ds(NBLK * G // 2, TAIL // 2)],
            )

    return body(tab_t, tail)


def _gather(pck, x2, pe):
    mesh = plsc.VectorSubcoreMesh(core_axis_name="c", subcore_axis_name="s")

    @functools.partial(
        pl.kernel,
        out_type=jax.ShapeDtypeStruct((SEQ, EMB, NB), jnp.float32),
        mesh=mesh,
        compiler_params=_PARAMS,
        scratch_types=[
            pltpu.VMEM((SEQ, G), jnp.int32),
            pltpu.VMEM((SEQ, EMB), jnp.float32),
            pltpu.VMEM((G,), jnp.int32),
            pltpu.VMEM((G,), jnp.int32),
            pltpu.VMEM((G,), jnp.int32),
            pltpu.VMEM((G,), jnp.int32),
            pltpu.VMEM((G, G), jnp.float32),
            pltpu.VMEM((G, G), jnp.float32),
            pltpu.VMEM((EMB, G), jnp.float32),
            pltpu.VMEM((EMB, G), jnp.float32),
            pltpu.SemaphoreType.DMA,
            pltpu.SemaphoreType.DMA,
            pltpu.SemaphoreType.DMA,
            pltpu.SemaphoreType.DMA,
        ],
    )
    def body(tab_hbm, x_hbm, pe_hbm, out_hbm, xb_v, pe_v,
             idx_a, idx_b, par_a, par_b, g_a, g_b, cell_a, cell_b,
             sga, sgb, swa, swb):
        wid = lax.axis_index("s") * NC + lax.axis_index("c")
        pltpu.sync_copy(pe_hbm, pe_v)
        pltpu.sync_copy(x_hbm.at[pl.ds(pl.multiple_of(wid * SEQ, 8), SEQ)], xb_v)
        iota = lax.iota(jnp.int32, L)
        base_b = pl.multiple_of(wid * G, G)

        def prep(s, idx_v, par_v, g_v, sem):
            for k in range(G // L):
                u = (s + SEQ * L * k) + SEQ * iota
                v = plsc.load_gather(xb_v, [u >> 7, u & 127])
                idx_v[pl.ds(k * L, L)] = v >> 1
                par_v[pl.ds(k * L, L)] = (v & 1) << 6
            pltpu.async_copy(tab_hbm.at[idx_v], g_v, sem)

        def compute(s, par_v, g_v, cell_v):
            cols_l = [k * L + iota for k in range(G // L)]
            par_l = [par_v[pl.ds(k * L, L)] for k in range(G // L)]
            pe_row = [pe_v[s, pl.ds(ei * L, L)] for ei in range(EMB // L)]

            @plsc.parallel_loop(0, L, 1, unroll=4)
            def _(t):
                perm = (iota + t) & (L - 1)
                for ei in range(EMB // L):
                    evec = perm + ei * L
                    sp = jnp.take_along_axis(pe_row[ei], perm, axis=0)
                    for ci in range(G // L):
                        val = plsc.load_gather(g_v, [cols_l[ci], par_l[ci] + evec])
                        plsc.store_scatter(cell_v, [evec, cols_l[ci]], val + sp)

        def wait_gather(g_v, sem):
            pltpu.make_async_copy(tab_hbm.at[idx_a], g_v, sem).wait()

        def start_write(s, cell_v, sem):
            pltpu.async_copy(cell_v, out_hbm.at[s, :, pl.ds(base_b, G)], sem)

        def wait_write(s, cell_v, sem):
            pltpu.make_async_copy(cell_v, out_hbm.at[s, :, pl.ds(base_b, G)], sem).wait()

        prep(0, idx_a, par_a, g_a, sga)

        def pipe(i, carry):
            sa = 2 * i
            prep(sa + 1, idx_b, par_b, g_b, sgb)
            wait_gather(g_a, sga)

            @pl.when(i != 0)
            def _():
                wait_write(sa - 2, cell_a, swa)

            compute(sa, par_a, g_a, cell_a)
            start_write(sa, cell_a, swa)
            prep(sa + 2, idx_a, par_a, g_a, sga)
            wait_gather(g_b, sgb)

            @pl.when(i != 0)
            def _():
                wait_write(sa - 1, cell_b, swb)

            compute(sa + 1, par_b, g_b, cell_b)
            start_write(sa + 1, cell_b, swb)
            return carry

        lax.fori_loop(0, SEQ // 2 - 1, pipe, 0)

        sa = SEQ - 2
        prep(sa + 1, idx_b, par_b, g_b, sgb)
        wait_gather(g_a, sga)
        wait_write(sa - 2, cell_a, swa)
        compute(sa, par_a, g_a, cell_a)
        start_write(sa, cell_a, swa)
        wait_gather(g_b, sgb)
        wait_write(sa - 1, cell_b, swb)
        compute(sa + 1, par_b, g_b, cell_b)
        start_write(sa + 1, cell_b, swb)
        wait_write(sa, cell_a, swa)
        wait_write(sa + 1, cell_b, swb)

    return body(pck, x2, pe)


def kernel(x, table, pos_enc):
    batch, seq = x.shape
    x2 = x.reshape(batch * seq // G, G)
    pck = _pack(table.T, table[NBLK * G :, :])
    out3 = _gather(pck, x2, pos_enc[:seq])
    return out3.transpose(2, 0, 1)


# speedup vs baseline: 6.0130x; 1.4200x over previous
import functools

import jax
import jax.numpy as jnp
from jax import lax
from jax.experimental import pallas as pl
from jax.experimental.pallas import tpu as pltpu
from jax.experimental.pallas import tpu_sc as plsc

EMB = 64
SEQ = 200
NB = 4096
VOCAB = 1000000
G = 128
L = 16
NC = 2
NS = 16
NW = NC * NS

NBLK = VOCAB // G
TAIL = VOCAB - NBLK * G

_PARAMS = pltpu.CompilerParams(
    use_tc_tiling_on_sc=True, needs_layout_passes=False
)


def _pack(tab_t, tail):
    mesh = plsc.VectorSubcoreMesh(core_axis_name="c", subcore_axis_name="s")

    @functools.partial(
        pl.kernel,
        out_type=jax.ShapeDtypeStruct((VOCAB // 2, 2 * EMB), jnp.float32),
        mesh=mesh,
        compiler_params=_PARAMS,
        scratch_types=[
            pltpu.VMEM((EMB, G), jnp.float32),
            pltpu.VMEM((EMB, G), jnp.float32),
            pltpu.VMEM((G // 2, 2 * EMB), jnp.float32),
            pltpu.VMEM((G // 2, 2 * EMB), jnp.float32),
            pltpu.VMEM((TAIL, EMB), jnp.float32),
            pltpu.SemaphoreType.DMA,
            pltpu.SemaphoreType.DMA,
            pltpu.SemaphoreType.DMA,
            pltpu.SemaphoreType.DMA,
        ],
    )
    def body(t_hbm, tail_hbm, out_hbm, blk_a, blk_b, pck_a, pck_b, tail_v,
             sla, slb, swa, swb):
        wid = lax.axis_index("s") * NC + lax.axis_index("c")
        iota = lax.iota(jnp.int32, L)
        cols_l = [cg * L + iota for cg in range(G // L)]
        rows_h = [(cg * L + iota) >> 1 for cg in range(G // L)]
        hvec = (iota & 1) * EMB

        def start_load(i, blk_v, sem):
            j = wid + NW * i
            pltpu.async_copy(
                t_hbm.at[:, pl.ds(pl.multiple_of(j * G, G), G)], blk_v, sem
            )

        def wait_load(i, blk_v, sem):
            j = wid + NW * i
            pltpu.make_async_copy(
                t_hbm.at[:, pl.ds(pl.multiple_of(j * G, G), G)], blk_v, sem
            ).wait()

        def compute(blk_v, pck_v):
            @plsc.parallel_loop(0, L, 1, unroll=4)
            def _(t):
                perm = (iota + t) & (L - 1)
                for ei in range(EMB // L):
                    evec = perm + ei * L
                    ecol = evec + hvec
                    for cg in range(G // L):
                        val = plsc.load_gather(blk_v, [evec, cols_l[cg]])
                        plsc.store_scatter(pck_v, [rows_h[cg], ecol], val)

        def out_slice(i):
            j = wid + NW * i
            return out_hbm.at[pl.ds(pl.multiple_of(j * (G // 2), 8), G // 2)]

        def start_write(i, pck_v, sem):
            pltpu.async_copy(pck_v, out_slice(i), sem)

        def wait_write(i, pck_v, sem):
            pltpu.make_async_copy(pck_v, out_slice(i), sem).wait()

        NCOM = NBLK // NW
        start_load(0, blk_a, sla)

        def pipe(p, carry):
            ia = 2 * p
            start_load(ia + 1, blk_b, slb)
            wait_load(ia, blk_a, sla)

            @pl.when(p != 0)
            def _():
                wait_write(ia - 2, pck_a, swa)

            compute(blk_a, pck_a)
            start_write(ia, pck_a, swa)
            start_load(ia + 2, blk_a, sla)
            wait_load(ia + 1, blk_b, slb)

            @pl.when(p != 0)
            def _():
                wait_write(ia - 1, pck_b, swb)

            compute(blk_b, pck_b)
            start_write(ia + 1, pck_b, swb)
            return carry

        lax.fori_loop(0, NCOM // 2 - 1, pipe, 0)

        ia = NCOM - 2
        start_load(ia + 1, blk_b, slb)
        wait_load(ia, blk_a, sla)
        wait_write(ia - 2, pck_a, swa)
        compute(blk_a, pck_a)
        start_write(ia, pck_a, swa)
        wait_load(ia + 1, blk_b, slb)
        wait_write(ia - 1, pck_b, swb)
        compute(blk_b, pck_b)
        start_write(ia + 1, pck_b, swb)
        wait_write(ia, pck_a, swa)
        wait_write(ia + 1, pck_b, swb)

        @pl.when(wid < NBLK - NCOM * NW)
        def _():
            pltpu.sync_copy(
                t_hbm.at[:, pl.ds(pl.multiple_of((wid + NW * NCOM) * G, G), G)],
                blk_a,
            )
            compute(blk_a, pck_a)
            pltpu.sync_copy(pck_a, out_slice(NCOM))

        @pl.when(wid == 0)
        def _():
            pltpu.sync_copy(tail_hbm, tail_v)

            def t_body(q, carry2):
                for d in range(2 * EMB // L):
                    r = 2 * q + d // (EMB // L)
                    pck_a[q, pl.ds(d * L, L)] = tail_v[
                        r, pl.ds((d % (EMB // L)) * L, L)
                    ]
                return carry2

            lax.fori_loop(0, TAIL // 2, t_body, 0)
            pltpu.sync_copy(
                pck_a.at[pl.ds(0, TAIL // 2)],
                out_hbm.at[pl.ds(NBLK * G // 2, TAIL // 2)],
            )

    return body(tab_t, tail)


def _gather(pck, x2, pe):
    mesh = plsc.VectorSubcoreMesh(core_axis_name="c", subcore_axis_name="s")

    @functools.partial(
        pl.kernel,
        out_type=jax.ShapeDtypeStruct((SEQ, EMB, NB), jnp.float32),
        mesh=mesh,
        compiler_params=_PARAMS,
        scratch_types=[
            pltpu.VMEM((SEQ, G), jnp.int32),
            pltpu.VMEM((SEQ, EMB), jnp.float32),
            pltpu.VMEM((G,), jnp.int32),
            pltpu.VMEM((G,), jnp.int32),
            pltpu.VMEM((G,), jnp.int32),
            pltpu.VMEM((G,), jnp.int32),
            pltpu.VMEM((G, G), jnp.float32),
            pltpu.VMEM((G, G), jnp.float32),
            pltpu.VMEM((EMB, G), jnp.float32),
            pltpu.VMEM((EMB, G), jnp.float32),
            pltpu.SemaphoreType.DMA,
            pltpu.SemaphoreType.DMA,
            pltpu.SemaphoreType.DMA,
            pltpu.SemaphoreType.DMA,
        ],
    )
    def body(tab_hbm, x_hbm, pe_hbm, out_hbm, xb_v, pe_v,
             idx_a, idx_b, par_a, par_b, g_a, g_b, cell_a, cell_b,
             sga, sgb, swa, swb):
        wid = lax.axis_index("s") * NC + lax.axis_index("c")
        pltpu.sync_copy(pe_hbm, pe_v)
        pltpu.sync_copy(x_hbm.at[pl.ds(pl.multiple_of(wid * SEQ, 8), SEQ)], xb_v)
        iota = lax.iota(jnp.int32, L)
        base_b = pl.multiple_of(wid * G, G)

        def prep(s, idx_v, par_v, g_v, sem):
            for k in range(G // L):
                u = (s + SEQ * L * k) + SEQ * iota
                v = plsc.load_gather(xb_v, [u >> 7, u & 127])
                idx_v[pl.ds(k * L, L)] = v >> 1
                par_v[pl.ds(k * L, L)] = (v & 1) << 6
            pltpu.async_copy(tab_hbm.at[idx_v], g_v, sem)

        def compute(s, par_v, g_v, cell_v):
            cols_l = [k * L + iota for k in range(G // L)]
            par_l = [par_v[pl.ds(k * L, L)] for k in range(G // L)]
            pe_row = [pe_v[s, pl.ds(ei * L, L)] for ei in range(EMB // L)]

            @plsc.parallel_loop(0, L, 1, unroll=4)
            def _(t):
                perm = (iota + t) & (L - 1)
                for ei in range(EMB // L):
                    evec = perm + ei * L
                    sp = jnp.take_along_axis(pe_row[ei], perm, axis=0)
                    for ci in range(G // L):
                        val = plsc.load_gather(g_v, [cols_l[ci], par_l[ci] + evec])
                        plsc.store_scatter(cell_v, [evec, cols_l[ci]], val + sp)

        def wait_gather(g_v, sem):
            pltpu.make_async_copy(tab_hbm.at[idx_a], g_v, sem).wait()

        def start_write(s, cell_v, sem):
            pltpu.async_copy(cell_v, out_hbm.at[s, :, pl.ds(base_b, G)], sem)

        def wait_write(s, cell_v, sem):
            pltpu.make_async_copy(cell_v, out_hbm.at[s, :, pl.ds(base_b, G)], sem).wait()

        prep(0, idx_a, par_a, g_a, sga)

        def pipe(i, carry):
            sa = 2 * i
            prep(sa + 1, idx_b, par_b, g_b, sgb)
            wait_gather(g_a, sga)

            @pl.when(i != 0)
            def _():
                wait_write(sa - 2, cell_a, swa)

            compute(sa, par_a, g_a, cell_a)
            start_write(sa, cell_a, swa)
            prep(sa + 2, idx_a, par_a, g_a, sga)
            wait_gather(g_b, sgb)

            @pl.when(i != 0)
            def _():
                wait_write(sa - 1, cell_b, swb)

            compute(sa + 1, par_b, g_b, cell_b)
            start_write(sa + 1, cell_b, swb)
            return carry

        lax.fori_loop(0, SEQ // 2 - 1, pipe, 0)

        sa = SEQ - 2
        prep(sa + 1, idx_b, par_b, g_b, sgb)
        wait_gather(g_a, sga)
        wait_write(sa - 2, cell_a, swa)
        compute(sa, par_a, g_a, cell_a)
        start_write(sa, cell_a, swa)
        wait_gather(g_b, sgb)
        wait_write(sa - 1, cell_b, swb)
        compute(sa + 1, par_b, g_b, cell_b)
        start_write(sa + 1, cell_b, swb)
        wait_write(sa, cell_a, swa)
        wait_write(sa + 1, cell_b, swb)

    return body(pck, x2, pe)


def kernel(x, table, pos_enc):
    batch, seq = x.shape
    x2 = x.reshape(batch * seq // G, G)
    pck = _pack(table.T, table[NBLK * G :, :])
    out3 = _gather(pck, x2, pos_enc[:seq])
    return out3.transpose(2, 0, 1)


# speedup vs baseline: 6.8907x; 1.1460x over previous
import functools

import jax
import jax.numpy as jnp
from jax import lax
from jax.experimental import pallas as pl
from jax.experimental.pallas import tpu as pltpu
from jax.experimental.pallas import tpu_sc as plsc

EMB = 64
SEQ = 200
NB = 4096
VOCAB = 1000000
G = 128
L = 16
NC = 2
NS = 16
NW = NC * NS

NBLK = VOCAB // G
TAIL = VOCAB - NBLK * G

_PARAMS = pltpu.CompilerParams(
    use_tc_tiling_on_sc=True, needs_layout_passes=False
)


def _pack(tab_t, tail):
    mesh = plsc.VectorSubcoreMesh(core_axis_name="c", subcore_axis_name="s")

    @functools.partial(
        pl.kernel,
        out_type=jax.ShapeDtypeStruct((VOCAB // 2, 2 * EMB), jnp.float32),
        mesh=mesh,
        compiler_params=_PARAMS,
        scratch_types=[
            pltpu.VMEM((EMB, G), jnp.float32),
            pltpu.VMEM((EMB, G), jnp.float32),
            pltpu.VMEM((G // 2, 2 * EMB), jnp.float32),
            pltpu.VMEM((G // 2, 2 * EMB), jnp.float32),
            pltpu.VMEM((TAIL, EMB), jnp.float32),
            pltpu.SemaphoreType.DMA,
            pltpu.SemaphoreType.DMA,
            pltpu.SemaphoreType.DMA,
            pltpu.SemaphoreType.DMA,
        ],
    )
    def body(t_hbm, tail_hbm, out_hbm, blk_a, blk_b, pck_a, pck_b, tail_v,
             sla, slb, swa, swb):
        wid = lax.axis_index("s") * NC + lax.axis_index("c")
        iota = lax.iota(jnp.int32, L)
        cols_l = [cg * L + iota for cg in range(G // L)]
        rows_h = [(cg * L + iota) >> 1 for cg in range(G // L)]
        hvec = (iota & 1) * EMB

        def start_load(i, blk_v, sem):
            j = wid + NW * i
            pltpu.async_copy(
                t_hbm.at[:, pl.ds(pl.multiple_of(j * G, G), G)], blk_v, sem
            )

        def wait_load(i, blk_v, sem):
            j = wid + NW * i
            pltpu.make_async_copy(
                t_hbm.at[:, pl.ds(pl.multiple_of(j * G, G), G)], blk_v, sem
            ).wait()

        def compute(blk_v, pck_v):
            @plsc.parallel_loop(0, L, 1, unroll=4)
            def _(t):
                perm = (iota + t) & (L - 1)
                for ei in range(EMB // L):
                    evec = perm + ei * L
                    ecol = evec + hvec
                    for cg in range(G // L):
                        val = plsc.load_gather(blk_v, [evec, cols_l[cg]])
                        plsc.store_scatter(pck_v, [rows_h[cg], ecol], val)

        def out_slice(i):
            j = wid + NW * i
            return out_hbm.at[pl.ds(pl.multiple_of(j * (G // 2), 8), G // 2)]

        def start_write(i, pck_v, sem):
            pltpu.async_copy(pck_v, out_slice(i), sem)

        def wait_write(i, pck_v, sem):
            pltpu.make_async_copy(pck_v, out_slice(i), sem).wait()

        NCOM = NBLK // NW
        start_load(0, blk_a, sla)

        def pipe(p, carry):
            ia = 2 * p
            start_load(ia + 1, blk_b, slb)
            wait_load(ia, blk_a, sla)

            @pl.when(p != 0)
            def _():
                wait_write(ia - 2, pck_a, swa)

            compute(blk_a, pck_a)
            start_write(ia, pck_a, swa)
            start_load(ia + 2, blk_a, sla)
            wait_load(ia + 1, blk_b, slb)

            @pl.when(p != 0)
            def _():
                wait_write(ia - 1, pck_b, swb)

            compute(blk_b, pck_b)
            start_write(ia + 1, pck_b, swb)
            return carry

        lax.fori_loop(0, NCOM // 2 - 1, pipe, 0)

        ia = NCOM - 2
        start_load(ia + 1, blk_b, slb)
        wait_load(ia, blk_a, sla)
        wait_write(ia - 2, pck_a, swa)
        compute(blk_a, pck_a)
        start_write(ia, pck_a, swa)
        wait_load(ia + 1, blk_b, slb)
        wait_write(ia - 1, pck_b, swb)
        compute(blk_b, pck_b)
        start_write(ia + 1, pck_b, swb)
        wait_write(ia, pck_a, swa)
        wait_write(ia + 1, pck_b, swb)

        @pl.when(wid < NBLK - NCOM * NW)
        def _():
            pltpu.sync_copy(
                t_hbm.at[:, pl.ds(pl.multiple_of((wid + NW * NCOM) * G, G), G)],
                blk_a,
            )
            compute(blk_a, pck_a)
            pltpu.sync_copy(pck_a, out_slice(NCOM))

        @pl.when(wid == 0)
        def _():
            pltpu.sync_copy(tail_hbm, tail_v)

            def t_body(q, carry2):
                for d in range(2 * EMB // L):
                    r = 2 * q + d // (EMB // L)
                    pck_a[q, pl.ds(d * L, L)] = tail_v[
                        r, pl.ds((d % (EMB // L)) * L, L)
                    ]
                return carry2

            lax.fori_loop(0, TAIL // 2, t_body, 0)
            pltpu.sync_copy(
                pck_a.at[pl.ds(0, TAIL // 2)],
                out_hbm.at[pl.ds(NBLK * G // 2, TAIL // 2)],
            )

    return body(tab_t, tail)


def _gather(pck, x2, pe):
    mesh = plsc.VectorSubcoreMesh(core_axis_name="c", subcore_axis_name="s")

    @functools.partial(
        pl.kernel,
        out_type=jax.ShapeDtypeStruct((SEQ, EMB, NB), jnp.float32),
        mesh=mesh,
        compiler_params=_PARAMS,
        scratch_types=[
            pltpu.VMEM((SEQ, G), jnp.int32),
            pltpu.VMEM((SEQ, EMB), jnp.float32),
            pltpu.VMEM((G,), jnp.int32),
            pltpu.VMEM((G,), jnp.int32),
            pltpu.VMEM((G,), jnp.int32),
            pltpu.VMEM((G,), jnp.int32),
            pltpu.VMEM((G, G), jnp.float32),
            pltpu.VMEM((G, G), jnp.float32),
            pltpu.VMEM((EMB, G), jnp.float32),
            pltpu.VMEM((EMB, G), jnp.float32),
            pltpu.SemaphoreType.DMA,
            pltpu.SemaphoreType.DMA,
            pltpu.SemaphoreType.DMA,
            pltpu.SemaphoreType.DMA,
        ],
    )
    def body(tab_hbm, x_hbm, pe_hbm, out_hbm, xb_v, pe_v,
             idx_a, idx_b, par_a, par_b, g_a, g_b, cell_a, cell_b,
             sga, sgb, swa, swb):
        wid = lax.axis_index("s") * NC + lax.axis_index("c")
        pltpu.sync_copy(pe_hbm, pe_v)
        pltpu.sync_copy(x_hbm.at[pl.ds(pl.multiple_of(wid * SEQ, 8), SEQ)], xb_v)
        iota = lax.iota(jnp.int32, L)
        base_b = pl.multiple_of(wid * G, G)

        def prep(s, idx_v, par_v, g_v, sem):
            for k in range(G // L):
                u = (s + SEQ * L * k) + SEQ * iota
                v = plsc.load_gather(xb_v, [u >> 7, u & 127])
                idx_v[pl.ds(k * L, L)] = v >> 1
                par_v[pl.ds(k * L, L)] = (v & 1) << 6
            pltpu.async_copy(tab_hbm.at[idx_v], g_v, sem)

        def compute(s, par_v, g_v, cell_v):
            cols_l = [k * L + iota for k in range(G // L)]
            par_l = [par_v[pl.ds(k * L, L)] for k in range(G // L)]
            pe_row = [pe_v[s, pl.ds(ei * L, L)] for ei in range(EMB // L)]

            @plsc.parallel_loop(0, L, 1, unroll=8)
            def _(t):
                perm = (iota + t) & (L - 1)
                for ei in range(EMB // L):
                    evec = perm + ei * L
                    sp = jnp.take_along_axis(pe_row[ei], perm, axis=0)
                    for ci in range(G // L):
                        val = plsc.load_gather(g_v, [cols_l[ci], par_l[ci] + evec])
                        plsc.store_scatter(cell_v, [evec, cols_l[ci]], val + sp)

        def wait_gather(g_v, sem):
            pltpu.make_async_copy(tab_hbm.at[idx_a], g_v, sem).wait()

        def start_write(s, cell_v, sem):
            pltpu.async_copy(cell_v, out_hbm.at[s, :, pl.ds(base_b, G)], sem)

        def wait_write(s, cell_v, sem):
            pltpu.make_async_copy(cell_v, out_hbm.at[s, :, pl.ds(base_b, G)], sem).wait()

        prep(0, idx_a, par_a, g_a, sga)

        def pipe(i, carry):
            sa = 2 * i
            prep(sa + 1, idx_b, par_b, g_b, sgb)
            wait_gather(g_a, sga)

            @pl.when(i != 0)
            def _():
                wait_write(sa - 2, cell_a, swa)

            compute(sa, par_a, g_a, cell_a)
            start_write(sa, cell_a, swa)
            prep(sa + 2, idx_a, par_a, g_a, sga)
            wait_gather(g_b, sgb)

            @pl.when(i != 0)
            def _():
                wait_write(sa - 1, cell_b, swb)

            compute(sa + 1, par_b, g_b, cell_b)
            start_write(sa + 1, cell_b, swb)
            return carry

        lax.fori_loop(0, SEQ // 2 - 1, pipe, 0)

        sa = SEQ - 2
        prep(sa + 1, idx_b, par_b, g_b, sgb)
        wait_gather(g_a, sga)
        wait_write(sa - 2, cell_a, swa)
        compute(sa, par_a, g_a, cell_a)
        start_write(sa, cell_a, swa)
        wait_gather(g_b, sgb)
        wait_write(sa - 1, cell_b, swb)
        compute(sa + 1, par_b, g_b, cell_b)
        start_write(sa + 1, cell_b, swb)
        wait_write(sa, cell_a, swa)
        wait_write(sa + 1, cell_b, swb)

    return body(pck, x2, pe)


def kernel(x, table, pos_enc):
    batch, seq = x.shape
    x2 = x.reshape(batch * seq // G, G)
    pck = _pack(table.T, table[NBLK * G :, :])
    out3 = _gather(pck, x2, pos_enc[:seq])
    return out3.transpose(2, 0, 1)
